# Initial kernel scaffold; baseline (speedup 1.0000x reference)
import jax
import jax.numpy as jnp
from jax.experimental import pallas as pl


def kernel(x, batch, weight, bias, mean_scale):
    raise NotImplementedError("write your pallas kernel here")


# speedup vs baseline: 3.3942x; 3.3942x over previous
import functools
import jax
import jax.numpy as jnp
from jax.experimental import pallas as pl
from jax.experimental.pallas import tpu as pltpu

N_NODES = 100000
N_FEAT = 128
N_GRAPHS = 512
EPS = 1e-05
BLK = 400
N_BLK = N_NODES // BLK


def _stats_body(idx_ref, x_ref, stats_ref):
    i = pl.program_id(0)

    @pl.when(i == 0)
    def _():
        stats_ref[...] = jnp.zeros_like(stats_ref)

    idx = idx_ref[0, 0, :]
    xb = x_ref[...]
    gids = jax.lax.broadcasted_iota(jnp.int32, (N_GRAPHS, BLK), 0)
    onehot = (gids == idx[None, :]).astype(jnp.float32)
    xcat = jnp.concatenate(
        [xb, xb * xb, jnp.ones((BLK, N_FEAT), jnp.float32)], axis=1
    )
    part = jax.lax.dot(onehot, xcat, preferred_element_type=jnp.float32)
    stats_ref[...] += part


def _apply_body(idx_ref, x_ref, stats_ref, msw_ref, bias_ref, out_ref, ab_ref):
    i = pl.program_id(0)

    @pl.when(i == 0)
    def _():
        stats = stats_ref[...]
        cnt = jnp.max(stats[:, 2 * N_FEAT:], axis=1, keepdims=True)
        c = jnp.maximum(cnt, 1.0)
        mean = stats[:, :N_FEAT] / c
        var = stats[:, N_FEAT:2 * N_FEAT] / c - mean * mean
        rstd = jax.lax.rsqrt(var + EPS)
        a = rstd * msw_ref[...]
        b = bias_ref[...] - mean * a
        ab_ref[...] = jnp.concatenate([a, b], axis=1)

    idx = idx_ref[0, 0, :]
    gids = jax.lax.broadcasted_iota(jnp.int32, (BLK, N_GRAPHS), 1)
    onehot = (gids == idx[:, None]).astype(jnp.float32)
    coef = jax.lax.dot(onehot, ab_ref[...], preferred_element_type=jnp.float32)
    out_ref[...] = x_ref[...] * coef[:, :N_FEAT] + coef[:, N_FEAT:]


@jax.jit
def kernel(x, batch, weight, bias, mean_scale):
    idx3 = batch.astype(jnp.int32).reshape(N_BLK, 1, BLK)
    stats = pl.pallas_call(
        _stats_body,
        grid=(N_BLK,),
        in_specs=[
            pl.BlockSpec((1, 1, BLK), lambda i: (i, 0, 0)),
            pl.BlockSpec((BLK, N_FEAT), lambda i: (i, 0)),
        ],
        out_specs=pl.BlockSpec((N_GRAPHS, 3 * N_FEAT), lambda i: (0, 0)),
        out_shape=jax.ShapeDtypeStruct((N_GRAPHS, 3 * N_FEAT), jnp.float32),
    )(idx3, x)

    msw = (mean_scale * weight).reshape(1, N_FEAT)
    bias2 = bias.reshape(1, N_FEAT)
    out = pl.pallas_call(
        _apply_body,
        grid=(N_BLK,),
        in_specs=[
            pl.BlockSpec((1, 1, BLK), lambda i: (i, 0, 0)),
            pl.BlockSpec((BLK, N_FEAT), lambda i: (i, 0)),
            pl.BlockSpec((N_GRAPHS, 3 * N_FEAT), lambda i: (0, 0)),
            pl.BlockSpec((1, N_FEAT), lambda i: (0, 0)),
            pl.BlockSpec((1, N_FEAT), lambda i: (0, 0)),
        ],
        out_specs=pl.BlockSpec((BLK, N_FEAT), lambda i: (i, 0)),
        out_shape=jax.ShapeDtypeStruct((N_NODES, N_FEAT), jnp.float32),
        scratch_shapes=[pltpu.VMEM((N_GRAPHS, 2 * N_FEAT), jnp.float32)],
    )(idx3, x, stats, msw, bias2)
    return out
